# baseline (speedup 1.0000x reference)
import functools

import jax
import jax.numpy as jnp
from jax import lax
from jax.experimental import pallas as pl
from jax.experimental.pallas import tpu as pltpu
from jax.experimental.pallas import tpu_sc as plsc

N = 100000
D = 128
CHUNK = 100
NCHUNK = N // CHUNK
NC, NS = 2, 16
NW = NC * NS
MAXI = -(-NCHUNK // NW)
NBUF = 2


def _body(idx_hbm, emb_hbm, out_hbm, *scr):
    table_sh = scr[0]
    idx_v = scr[1:1 + MAXI]
    rows_v = scr[1 + MAXI:1 + MAXI + NBUF]
    sem_i = scr[1 + MAXI + NBUF]
    sem_g = scr[2 + MAXI + NBUF:2 + MAXI + 2 * NBUF]
    sem_o = scr[2 + MAXI + 2 * NBUF:2 + MAXI + 3 * NBUF]

    c = lax.axis_index("c")
    s = lax.axis_index("s")
    wid = s * NC + c

    def guard(i):
        return wid + i * NW < NCHUNK

    idx_copies = {}
    for i in range(MAXI):
        k = wid + i * NW

        @pl.when(guard(i))
        def _fetch(i=i, k=k):
            idx_copies[i] = pltpu.async_copy(idx_hbm.at[k], idx_v[i], sem_i)

    @pl.when(s == 0)
    def _stage():
        pltpu.sync_copy(emb_hbm, table_sh.at[pl.ds(1, 36)])

    plsc.subcore_barrier()

    for i in range(MAXI):
        @pl.when(guard(i))
        def _drain_idx(i=i):
            idx_copies[i].wait()

    gathers = {}
    outs = {}

    for i in range(NBUF - 1):
        @pl.when(guard(i))
        def _prime(i=i):
            gathers[i] = pltpu.async_copy(
                table_sh.at[idx_v[i]], rows_v[i % NBUF], sem_g[i % NBUF])

    for i in range(MAXI):
        k = wid + i * NW
        j = i + NBUF - 1

        if j < MAXI:
            @pl.when(guard(j))
            def _issue_next(i=i, j=j):
                b = j % NBUF
                if j - NBUF >= 0:
                    outs[j - NBUF].wait()
                gathers[j] = pltpu.async_copy(
                    table_sh.at[idx_v[j]], rows_v[b], sem_g[b])

        @pl.when(guard(i))
        def _finish(i=i, k=k):
            gathers[i].wait()
            outs[i] = pltpu.async_copy(rows_v[i % NBUF], out_hbm.at[k],
                                       sem_o[i % NBUF])

    for i in range(max(0, MAXI - NBUF - 1), MAXI):
        k = wid + i * NW

        @pl.when((k < NCHUNK) & (k + NBUF * NW >= NCHUNK))
        def _drain(i=i):
            outs[i].wait()


@jax.jit
def _embed(idx2, embedding):
    mesh = plsc.VectorSubcoreMesh(core_axis_name="c", subcore_axis_name="s")
    f = functools.partial(
        pl.kernel,
        out_type=jax.ShapeDtypeStruct((NCHUNK, CHUNK, D), jnp.float32),
        mesh=mesh,
        scratch_types=[
            pltpu.VMEM_SHARED((37, D), jnp.float32),
        ] + [pltpu.VMEM((CHUNK,), jnp.int32)] * MAXI
          + [pltpu.VMEM((CHUNK, D), jnp.float32)] * NBUF
          + [pltpu.SemaphoreType.DMA] * (1 + 2 * NBUF),
    )(_body)
    return f(idx2, embedding)


def kernel(inputs, embedding):
    out = _embed(inputs.reshape(NCHUNK, CHUNK), embedding)
    return out.reshape(N, D)


# speedup vs baseline: 2.1800x; 2.1800x over previous
import functools

import jax
import jax.numpy as jnp
from jax import lax
from jax.experimental import pallas as pl
from jax.experimental.pallas import tpu as pltpu
from jax.experimental.pallas import tpu_sc as plsc

N = 100000
D = 128
CHUNK = 200
NCHUNK = N // CHUNK
NC, NS = 2, 16
NW = NC * NS
MAXI = -(-NCHUNK // NW)
NBUF = 4


def _body(idx_hbm, emb_hbm, out_hbm, *scr):
    table_sh = scr[0]
    idx_v = scr[1:1 + MAXI]
    rows_v = scr[1 + MAXI:1 + MAXI + NBUF]
    sem_i = scr[1 + MAXI + NBUF]
    sem_g = scr[2 + MAXI + NBUF:2 + MAXI + 2 * NBUF]
    sem_o = scr[2 + MAXI + 2 * NBUF:2 + MAXI + 3 * NBUF]

    c = lax.axis_index("c")
    s = lax.axis_index("s")
    wid = s * NC + c

    def guard(i):
        return wid + i * NW < NCHUNK

    idx_copies = {}
    for i in range(MAXI):
        k = wid + i * NW

        @pl.when(guard(i))
        def _fetch(i=i, k=k):
            idx_copies[i] = pltpu.async_copy(idx_hbm.at[k], idx_v[i], sem_i)

    @pl.when(s == 0)
    def _stage():
        pltpu.sync_copy(emb_hbm, table_sh.at[pl.ds(1, 36)])

    plsc.subcore_barrier()

    for i in range(MAXI):
        @pl.when(guard(i))
        def _drain_idx(i=i):
            idx_copies[i].wait()

    gathers = {}
    outs = {}

    for i in range(NBUF - 1):
        @pl.when(guard(i))
        def _prime(i=i):
            gathers[i] = pltpu.async_copy(
                table_sh.at[idx_v[i]], rows_v[i % NBUF], sem_g[i % NBUF])

    for i in range(MAXI):
        k = wid + i * NW
        j = i + NBUF - 1

        if j < MAXI:
            @pl.when(guard(j))
            def _issue_next(i=i, j=j):
                b = j % NBUF
                if j - NBUF >= 0:
                    outs[j - NBUF].wait()
                gathers[j] = pltpu.async_copy(
                    table_sh.at[idx_v[j]], rows_v[b], sem_g[b])

        @pl.when(guard(i))
        def _finish(i=i, k=k):
            gathers[i].wait()
            outs[i] = pltpu.async_copy(rows_v[i % NBUF], out_hbm.at[k],
                                       sem_o[i % NBUF])

    for i in range(max(0, MAXI - NBUF - 1), MAXI):
        k = wid + i * NW

        @pl.when((k < NCHUNK) & (k + NBUF * NW >= NCHUNK))
        def _drain(i=i):
            outs[i].wait()


@jax.jit
def _embed(idx2, embedding):
    mesh = plsc.VectorSubcoreMesh(core_axis_name="c", subcore_axis_name="s")
    f = functools.partial(
        pl.kernel,
        out_type=jax.ShapeDtypeStruct((NCHUNK, CHUNK, D), jnp.float32),
        mesh=mesh,
        scratch_types=[
            pltpu.VMEM_SHARED((37, D), jnp.float32),
        ] + [pltpu.VMEM((CHUNK,), jnp.int32)] * MAXI
          + [pltpu.VMEM((CHUNK, D), jnp.float32)] * NBUF
          + [pltpu.SemaphoreType.DMA] * (1 + 2 * NBUF),
    )(_body)
    return f(idx2, embedding)


def kernel(inputs, embedding):
    out = _embed(inputs.reshape(NCHUNK, CHUNK), embedding)
    return out.reshape(N, D)
